# baseline (speedup 1.0000x reference)
import jax
import jax.numpy as jnp
from jax.experimental import pallas as pl
from jax.experimental.pallas import tpu as pltpu

N = 8192
TD = 768
ID = 768
H = 512
OD = 768
E = 8
NOISE_STD = 1.0

T = 512


def _moe_fused_body(xt_ref, xi_ref, wt_ref, bt_ref, wi_ref, bi_ref,
                    wg_ref, bg_ref, noise_ref, w1_ref, b1_ref, w2_ref, b2_ref,
                    out_ref, w1bf_ref, w2bf_ref, hg_ref, wtbf_ref, wibf_ref):
    t = pl.program_id(0)

    @pl.when(t == 0)
    def _():
        w1bf_ref[...] = w1_ref[...].astype(jnp.bfloat16)
        w2bf_ref[...] = w2_ref[...].reshape(E * H, OD).astype(jnp.bfloat16)
        wtbf_ref[...] = wt_ref[...].astype(jnp.bfloat16)
        wibf_ref[...] = wi_ref[...].astype(jnp.bfloat16)

    tp = jnp.dot(xt_ref[...].astype(jnp.bfloat16), wtbf_ref[...],
                 preferred_element_type=jnp.float32)
    tp = tp + bt_ref[...]
    ip = jnp.dot(xi_ref[...].astype(jnp.bfloat16), wibf_ref[...],
                 preferred_element_type=jnp.float32)
    ip = ip + bi_ref[...]
    comb = jnp.concatenate([tp, ip], axis=1)

    logits = jnp.dot(comb, wg_ref[...], preferred_element_type=jnp.float32)
    logits = logits + bg_ref[...] + noise_ref[...] * NOISE_STD

    lane = jax.lax.broadcasted_iota(jnp.int32, (T, E), 1)
    m1 = jnp.max(logits, axis=1, keepdims=True)
    is1 = logits == m1
    idx1 = jnp.min(jnp.where(is1, lane, E), axis=1, keepdims=True)
    masked = jnp.where(lane == idx1, -jnp.inf, logits)
    m2 = jnp.max(masked, axis=1, keepdims=True)
    is2 = masked == m2
    idx2 = jnp.min(jnp.where(is2, lane, E), axis=1, keepdims=True)
    z = jnp.exp(m2 - m1)
    w1 = 1.0 / (1.0 + z)
    w2 = 1.0 - w1
    gates = jnp.where(lane == idx1, w1, jnp.where(lane == idx2, w2, 0.0))

    x = comb.astype(jnp.bfloat16)
    EH = E // 2
    y = jnp.dot(gates, b2_ref[...], preferred_element_type=jnp.float32)
    for half in range(2):
        for k in range(EH):
            e = half * EH + k
            he = jnp.dot(x, w1bf_ref[e], preferred_element_type=jnp.float32)
            ge = jnp.sum(jnp.where(lane == e, gates, 0.0), axis=1,
                         keepdims=True)
            hg_ref[:, k * H:(k + 1) * H] = (
                jnp.maximum(he + b1_ref[e], 0.0) * ge).astype(jnp.bfloat16)
        y = y + jnp.dot(hg_ref[...],
                        w2bf_ref[pl.ds(half * EH * H, EH * H), :],
                        preferred_element_type=jnp.float32)
    out_ref[...] = y


def kernel(text_emb, image_emb, Wt, bt, Wi, bi, Wg, bg, W1, b1, W2, b2, noise):
    out = pl.pallas_call(
        _moe_fused_body,
        grid=(N // T,),
        in_specs=[
            pl.BlockSpec((T, TD), lambda t: (t, 0)),
            pl.BlockSpec((T, ID), lambda t: (t, 0)),
            pl.BlockSpec((TD, H), lambda t: (0, 0)),
            pl.BlockSpec((H,), lambda t: (0,)),
            pl.BlockSpec((ID, H), lambda t: (0, 0)),
            pl.BlockSpec((H,), lambda t: (0,)),
            pl.BlockSpec((2 * H, E), lambda t: (0, 0)),
            pl.BlockSpec((E,), lambda t: (0,)),
            pl.BlockSpec((T, E), lambda t: (t, 0)),
            pl.BlockSpec((E, 2 * H, H), lambda t: (0, 0, 0)),
            pl.BlockSpec((E, H), lambda t: (0, 0)),
            pl.BlockSpec((E, H, OD), lambda t: (0, 0, 0)),
            pl.BlockSpec((E, OD), lambda t: (0, 0)),
        ],
        out_specs=pl.BlockSpec((T, OD), lambda t: (t, 0)),
        out_shape=jax.ShapeDtypeStruct((N, OD), jnp.float32),
        scratch_shapes=[
            pltpu.VMEM((E, 2 * H, H), jnp.bfloat16),
            pltpu.VMEM((E * H, OD), jnp.bfloat16),
            pltpu.VMEM((T, E * H // 2), jnp.bfloat16),
            pltpu.VMEM((TD, H), jnp.bfloat16),
            pltpu.VMEM((ID, H), jnp.bfloat16),
        ],
        compiler_params=pltpu.CompilerParams(
            dimension_semantics=("arbitrary",),
            vmem_limit_bytes=63 * 1024 * 1024),
    )(text_emb, image_emb, Wt, bt, Wi, bi, Wg, bg, noise, W1, b1, W2, b2)
    return out


# speedup vs baseline: 1.0020x; 1.0020x over previous
import jax
import jax.numpy as jnp
from jax.experimental import pallas as pl
from jax.experimental.pallas import tpu as pltpu

N = 8192
TD = 768
ID = 768
H = 512
OD = 768
E = 8
NOISE_STD = 1.0

T = 512


def _moe_fused_body(xt_ref, xi_ref, wt_ref, bt_ref, wi_ref, bi_ref,
                    wg_ref, bg_ref, noise_ref, w1_ref, b1_ref, w2_ref, b2_ref,
                    out_ref, w1bf_ref, w2bf_ref, hg_ref):
    t = pl.program_id(0)

    @pl.when(t == 0)
    def _():
        for ee in range(E):
            w1bf_ref[ee // 4, :, (ee % 4) * H:(ee % 4 + 1) * H] = (
                w1_ref[ee].astype(jnp.bfloat16))
        w2bf_ref[...] = w2_ref[...].reshape(E * H, OD).astype(jnp.bfloat16)

    tp = jnp.dot(xt_ref[...], wt_ref[...], preferred_element_type=jnp.float32)
    tp = tp + bt_ref[...]
    ip = jnp.dot(xi_ref[...], wi_ref[...], preferred_element_type=jnp.float32)
    ip = ip + bi_ref[...]
    comb = jnp.concatenate([tp, ip], axis=1)

    logits = jnp.dot(comb, wg_ref[...], preferred_element_type=jnp.float32)
    logits = logits + bg_ref[...] + noise_ref[...] * NOISE_STD

    lane = jax.lax.broadcasted_iota(jnp.int32, (T, E), 1)
    m1 = jnp.max(logits, axis=1, keepdims=True)
    is1 = logits == m1
    idx1 = jnp.min(jnp.where(is1, lane, E), axis=1, keepdims=True)
    masked = jnp.where(lane == idx1, -jnp.inf, logits)
    m2 = jnp.max(masked, axis=1, keepdims=True)
    is2 = masked == m2
    idx2 = jnp.min(jnp.where(is2, lane, E), axis=1, keepdims=True)
    z = jnp.exp(m2 - m1)
    w1 = 1.0 / (1.0 + z)
    w2 = 1.0 - w1
    gates = jnp.where(lane == idx1, w1, jnp.where(lane == idx2, w2, 0.0))

    x = comb.astype(jnp.bfloat16)
    EH = E // 2
    y = jnp.dot(gates, b2_ref[...], preferred_element_type=jnp.float32)
    for half in range(2):
        he_all = jnp.dot(x, w1bf_ref[half],
                         preferred_element_type=jnp.float32)
        for k in range(EH):
            e = half * EH + k
            ge = jnp.sum(jnp.where(lane == e, gates, 0.0), axis=1,
                         keepdims=True)
            hg_ref[:, k * H:(k + 1) * H] = (
                jnp.maximum(he_all[:, k * H:(k + 1) * H] + b1_ref[e], 0.0)
                * ge).astype(jnp.bfloat16)
        y = y + jnp.dot(hg_ref[...],
                        w2bf_ref[pl.ds(half * EH * H, EH * H), :],
                        preferred_element_type=jnp.float32)
    out_ref[...] = y


def kernel(text_emb, image_emb, Wt, bt, Wi, bi, Wg, bg, W1, b1, W2, b2, noise):
    out = pl.pallas_call(
        _moe_fused_body,
        grid=(N // T,),
        in_specs=[
            pl.BlockSpec((T, TD), lambda t: (t, 0)),
            pl.BlockSpec((T, ID), lambda t: (t, 0)),
            pl.BlockSpec((TD, H), lambda t: (0, 0)),
            pl.BlockSpec((H,), lambda t: (0,)),
            pl.BlockSpec((ID, H), lambda t: (0, 0)),
            pl.BlockSpec((H,), lambda t: (0,)),
            pl.BlockSpec((2 * H, E), lambda t: (0, 0)),
            pl.BlockSpec((E,), lambda t: (0,)),
            pl.BlockSpec((T, E), lambda t: (t, 0)),
            pl.BlockSpec((E, 2 * H, H), lambda t: (0, 0, 0)),
            pl.BlockSpec((E, H), lambda t: (0, 0)),
            pl.BlockSpec((E, H, OD), lambda t: (0, 0, 0)),
            pl.BlockSpec((E, OD), lambda t: (0, 0)),
        ],
        out_specs=pl.BlockSpec((T, OD), lambda t: (t, 0)),
        out_shape=jax.ShapeDtypeStruct((N, OD), jnp.float32),
        scratch_shapes=[
            pltpu.VMEM((2, 2 * H, E * H // 2), jnp.bfloat16),
            pltpu.VMEM((E * H, OD), jnp.bfloat16),
            pltpu.VMEM((T, E * H // 2), jnp.bfloat16),
        ],
        compiler_params=pltpu.CompilerParams(
            dimension_semantics=("arbitrary",),
            vmem_limit_bytes=63 * 1024 * 1024),
    )(text_emb, image_emb, Wt, bt, Wi, bi, Wg, bg, noise, W1, b1, W2, b2)
    return out
